# baseline (speedup 1.0000x reference)
import jax
import jax.numpy as jnp
from jax import lax
from jax.experimental import pallas as pl
from jax.experimental.pallas import tpu as pltpu
from jax.experimental.pallas import tpu_sc as plsc

_N = 16384
_D = 64
_NW = 32
_COLS_PER_W = _N // _NW
_CHUNKS = _COLS_PER_W // 16


def _sc_body(x_hbm, out_hbm, gu_v, gi_v, out_v):
    wid = lax.axis_index("s") * 2 + lax.axis_index("c")
    c0 = wid * _COLS_PER_W
    pltpu.sync_copy(x_hbm.at[0, :, pl.ds(c0, _COLS_PER_W)], gu_v)
    pltpu.sync_copy(x_hbm.at[1, :, pl.ds(c0, _COLS_PER_W)], gi_v)

    def chunk(s, carry):
        off = s * 16
        acc = jnp.zeros((16,), jnp.float32)
        for k in range(_D):
            a = gu_v[k, pl.ds(off, 16)]
            b = gi_v[k, pl.ds(off, 16)]
            acc = acc + a * b
        out_v[pl.ds(off, 16)] = acc
        return carry

    lax.fori_loop(0, _CHUNKS, chunk, 0)
    pltpu.sync_copy(out_v, out_hbm.at[pl.ds(c0, _COLS_PER_W)])


def _sc_rowdot(x):
    mesh = plsc.VectorSubcoreMesh(core_axis_name="c", subcore_axis_name="s")
    return pl.kernel(
        _sc_body,
        mesh=mesh,
        out_type=jax.ShapeDtypeStruct((_N,), jnp.float32),
        scratch_types=[
            pltpu.VMEM((_D, _COLS_PER_W), jnp.float32),
            pltpu.VMEM((_D, _COLS_PER_W), jnp.float32),
            pltpu.VMEM((_COLS_PER_W,), jnp.float32),
        ],
    )(x)


def kernel(inputs):
    t = jnp.swapaxes(inputs, 1, 2)
    return _sc_rowdot(t)


# speedup vs baseline: 5.7080x; 5.7080x over previous
import jax
import jax.numpy as jnp
from jax.experimental import pallas as pl
from jax.experimental.pallas import tpu as pltpu


def _rowdot_kernel(gu_ref, gi_ref, out_ref, acc_ref):
    i = pl.program_id(0)
    p = gu_ref[0] * gi_ref[0]
    p4 = p[0:8] + p[8:16] + p[16:24] + p[24:32]

    @pl.when(i == 0)
    def _init():
        acc_ref[...] = p4

    @pl.when(i != 0)
    def _accum():
        acc_ref[...] += p4

    @pl.when(i == pl.num_programs(0) - 1)
    def _finish():
        out_ref[...] = jnp.sum(acc_ref[...], axis=0)


def kernel(inputs):
    n = inputs.shape[1]
    d = inputs.shape[2]
    t = jnp.swapaxes(inputs, 1, 2)
    kblock = 32
    return pl.pallas_call(
        _rowdot_kernel,
        grid=(d // kblock,),
        in_specs=[
            pl.BlockSpec((1, kblock, n), lambda i: (0, i, 0)),
            pl.BlockSpec((1, kblock, n), lambda i: (1, i, 0)),
        ],
        out_specs=pl.BlockSpec((n,), lambda i: (0,)),
        out_shape=jax.ShapeDtypeStruct((n,), inputs.dtype),
        scratch_shapes=[pltpu.VMEM((8, n), jnp.float32)],
        compiler_params=pltpu.CompilerParams(
            dimension_semantics=("arbitrary",),
        ),
    )(t, t)
